# baseline (speedup 1.0000x reference)
import functools

import jax
import jax.numpy as jnp
from jax import lax
from jax.experimental import pallas as pl
from jax.experimental.pallas import tpu as pltpu
import jax.experimental.pallas.tpu_sc as plsc

F32 = jnp.float32
FOLD = 8


def _init_body(xin, vW, vb, eW8, eb8, M8r, ea, x0_out, w0_out, *, RTF, U):
    t = pl.program_id(0)

    @pl.when(t == 0)
    def _():
        x0_out[...] = jax.nn.silu(
            jnp.dot(xin[...], vW[...], preferred_element_type=F32) + vb[...])

    eaf = jnp.dot(ea[...], M8r[...], preferred_element_type=F32)
    w0_out[...] = jax.nn.silu(eaf * eW8[...] + eb8[...])


def _xproj_body(x_ref, w1r, b1r, w2r, b2r, w3r, b3r, w4r, b4r,
                x1_out, x3_out, t2_out, t4_out):
    x = x_ref[...]
    x1_out[...] = jnp.dot(x, w1r[...], preferred_element_type=F32) + b1r[...]
    x3_out[...] = jnp.dot(x, w3r[...], preferred_element_type=F32) + b3r[...]
    t2_out[...] = jnp.dot(x, w2r[...], preferred_element_type=F32) + b2r[...]
    t4_out[...] = jnp.dot(x, w4r[...], preferred_element_type=F32) + b4r[...]


def _make_gather(BGS, U):
    NC, NS = 2, 16
    NW = NC * NS
    EPW = BGS // NW
    CH = 128
    NCH = EPW // CH
    NB = 4
    NGRP = NCH // NB
    SRPC = CH // FOLD
    UL = FOLD * U
    FR = BGS // FOLD
    mesh = plsc.VectorSubcoreMesh(core_axis_name="c", subcore_axis_name="s",
                                  num_cores=NC, num_subcores=NS)

    out_ty = jax.ShapeDtypeStruct((BGS, U), F32)
    scratch = [pltpu.VMEM((NCH, CH), jnp.int32)]
    scratch += [pltpu.VMEM((CH, U), F32) for _ in range(2 * NB)]
    scratch += [pltpu.SemaphoreType.DMA for _ in range(2 * NB)]

    @functools.partial(
        pl.kernel,
        out_type=[out_ty, out_ty],
        mesh=mesh,
        scratch_types=scratch,
        compiler_params=pltpu.CompilerParams(use_tc_tiling_on_sc=False),
    )
    def gather_k(t2_hbm, t4_hbm, idx_hbm, g2_hbm, g4_hbm, idx_v, *rest):
        r2 = rest[0:NB]
        r4 = rest[NB:2 * NB]
        gsem = rest[2 * NB:3 * NB]
        osem = rest[3 * NB:4 * NB]
        wid = lax.axis_index("s") * NC + lax.axis_index("c")
        obase = wid * EPW
        pltpu.sync_copy(idx_hbm.at[pl.ds(wid * NCH, NCH)], idx_v)

        def gath(c, b):
            pltpu.async_copy(t2_hbm.at[idx_v.at[c]], r2[b], gsem[b])
            pltpu.async_copy(t4_hbm.at[idx_v.at[c]], r4[b], gsem[b])

        def wait_g(b):
            pltpu.make_async_copy(t2_hbm.at[idx_v.at[0]], r2[b], gsem[b]).wait()
            pltpu.make_async_copy(t4_hbm.at[idx_v.at[0]], r4[b], gsem[b]).wait()

        g2f = g2_hbm
        g4f = g4_hbm

        def put(c, b):
            o = obase + c * CH
            pltpu.async_copy(r2[b], g2f.at[pl.ds(o, CH)], osem[b])
            pltpu.async_copy(r4[b], g4f.at[pl.ds(o, CH)], osem[b])

        def wait_o(b):
            pltpu.make_async_copy(r2[b], g2f.at[pl.ds(0, CH)],
                                  osem[b]).wait()
            pltpu.make_async_copy(r4[b], g4f.at[pl.ds(0, CH)],
                                  osem[b]).wait()

        for b in range(NB):
            gath(b, b)

        def grp_body(g, carry):
            for b in range(NB):
                c = g * NB + b
                wait_g(b)
                put(c, b)
                wait_o(b)
                gath(c + NB, b)
            return carry

        lax.fori_loop(0, NGRP - 1, grp_body, 0)
        for b in range(NB):
            wait_g(b)
            put((NGRP - 1) * NB + b, b)
        for b in range(NB):
            wait_o(b)

    return gather_k


def _stats_body(w_ref, g2_ref, g4_ref, x_ref, x1_ref, x3_ref,
                E8r, b8r, Kr, K2r, vgr, vbr, egr, ebr,
                w1n, b1n, w2n, b2n, w3n, b3n, w4n, b4n,
                u_out, scl_out, xn_out, x1n_out, x3n_out, t2n_out, t4n_out,
                agg, s1, s2, *, RTF, S, U, BGS, BG, T):
    t = pl.program_id(0)
    UL = FOLD * U
    NPT = RTF * FOLD // S
    SPF = S // FOLD

    @pl.when(t == 0)
    def _():
        s1[...] = jnp.zeros_like(s1)
        s2[...] = jnp.zeros_like(s2)

    w0 = w_ref[...]
    x3t = x3_ref[pl.ds(t * NPT, NPT), :]
    x3f8 = jnp.dot(x3t, Kr[...], preferred_element_type=F32)
    x3f = jnp.broadcast_to(x3f8[:, None, :],
                           (NPT, SPF, UL)).reshape(RTF, UL)
    w1 = jnp.dot(w0, E8r[...], preferred_element_type=F32) + b8r[...]
    u = w1 + x3f + g4_ref[...]
    s1[...] += jnp.sum(u, axis=0)[None, :]
    s2[...] += jnp.sum(u * u, axis=0)[None, :]
    u_out[...] = u.astype(jnp.bfloat16)
    pr = jax.nn.sigmoid(w0) * g2_ref[...]
    pa = pr.reshape(NPT, SPF, UL).sum(axis=1)
    agg[pl.ds(t * NPT, NPT), :] = jnp.dot(
        pa, K2r[...], preferred_element_type=F32) / S

    @pl.when(t == T - 1)
    def _final():
        s1f = jnp.dot(s1[...], K2r[...], preferred_element_type=F32)
        s2f = jnp.dot(s2[...], K2r[...], preferred_element_type=F32)
        mean = s1f / BGS
        var = s2f / BGS - mean * mean
        inv = lax.rsqrt(var + 1e-5)
        sc48 = egr[...] * inv
        sh48 = ebr[...] - mean * sc48
        scl_out[...] = jnp.concatenate(
            [jnp.dot(sc48, Kr[...], preferred_element_type=F32),
             jnp.dot(sh48, Kr[...], preferred_element_type=F32)], axis=0)
        tv = x1_ref[...] + agg[...]
        mu = jnp.mean(tv, axis=0, keepdims=True)
        varv = jnp.mean(tv * tv, axis=0, keepdims=True) - mu * mu
        invv = lax.rsqrt(varv + 1e-5)
        xn = x_ref[...] + jax.nn.silu(
            (tv - mu) * invv * vgr[...] + vbr[...])
        xn_out[...] = xn
        x1n_out[...] = jnp.dot(xn, w1n[...], preferred_element_type=F32) + b1n[...]
        x3n_out[...] = jnp.dot(xn, w3n[...], preferred_element_type=F32) + b3n[...]
        t2n_out[...] = jnp.dot(xn, w2n[...], preferred_element_type=F32) + b2n[...]
        t4n_out[...] = jnp.dot(xn, w4n[...], preferred_element_type=F32) + b4n[...]


def _apply_body(w_ref, u_ref, scl_ref, wn_ref):
    u = u_ref[...].astype(F32)
    wn_ref[...] = w_ref[...] + jax.nn.silu(
        u * scl_ref[0:1, :] + scl_ref[1:2, :])


def kernel(x, edge_index, edge_attr, v_lin0_W, v_lin0_b, v1_W, v1_b,
           v2_W, v2_b, v3_W, v3_b, v4_W, v4_b, vbn_g, vbn_b,
           e_lin0_W, e_lin0_b, e0_W, e0_b, ebn_g, ebn_b):
    B, G, S = edge_index.shape
    U = v1_W.shape[-1]
    DEPTH = v1_W.shape[0]
    BG = B * G
    BGS = BG * S
    UL = FOLD * U
    FR = BGS // FOLD
    T = 16
    RTF = FR // T

    xin = x.reshape(BG, x.shape[-1])
    ea_f = edge_attr.reshape(FR, FOLD)
    flat_idx = (edge_index.astype(jnp.int32)
                + (jnp.arange(B, dtype=jnp.int32) * G)[:, None, None])
    flat_idx = flat_idx.reshape(BGS // 128, 128)

    r1 = lambda a: a.reshape(1, -1)
    tile8 = lambda a: jnp.tile(a.reshape(1, -1), (1, FOLD))
    eye8 = jnp.eye(FOLD, dtype=F32)

    init_call = pl.pallas_call(
        functools.partial(_init_body, RTF=RTF, U=U),
        grid=(T,),
        in_specs=[
            pl.BlockSpec((BG, xin.shape[1]), lambda t: (0, 0)),
            pl.BlockSpec(v_lin0_W.shape, lambda t: (0, 0)),
            pl.BlockSpec((1, U), lambda t: (0, 0)),
            pl.BlockSpec((1, UL), lambda t: (0, 0)),
            pl.BlockSpec((1, UL), lambda t: (0, 0)),
            pl.BlockSpec((FOLD, UL), lambda t: (0, 0)),
            pl.BlockSpec((RTF, FOLD), lambda t: (t, 0)),
        ],
        out_specs=[
            pl.BlockSpec((BG, U), lambda t: (0, 0)),
            pl.BlockSpec((RTF, UL), lambda t: (t, 0)),
        ],
        out_shape=[
            jax.ShapeDtypeStruct((BG, U), F32),
            jax.ShapeDtypeStruct((FR, UL), F32),
        ],
        compiler_params=pltpu.CompilerParams(
            dimension_semantics=("arbitrary",)),
    )
    M8 = jnp.kron(eye8, jnp.ones((1, U), dtype=F32))
    K = jnp.kron(jnp.ones((1, FOLD), dtype=F32), jnp.eye(U, dtype=F32))
    K2 = jnp.kron(jnp.ones((FOLD, 1), dtype=F32), jnp.eye(U, dtype=F32))
    xc, w = init_call(xin, v_lin0_W, r1(v_lin0_b),
                      tile8(e_lin0_W), tile8(e_lin0_b), M8, ea_f)

    xproj_call = pl.pallas_call(
        _xproj_body,
        out_shape=[jax.ShapeDtypeStruct((BG, U), F32) for _ in range(4)],
    )

    gather_call = _make_gather(BGS, U)

    full = lambda shp: pl.BlockSpec(shp, lambda t: (0, 0))
    stats_call = pl.pallas_call(
        functools.partial(_stats_body, RTF=RTF, S=S, U=U, BGS=BGS, BG=BG, T=T),
        grid=(T,),
        in_specs=[
            pl.BlockSpec((RTF, UL), lambda t: (t, 0)),
            pl.BlockSpec((RTF, UL), lambda t: (t, 0)),
            pl.BlockSpec((RTF, UL), lambda t: (t, 0)),
            full((BG, U)), full((BG, U)), full((BG, U)),
            full((UL, UL)), full((1, UL)), full((U, UL)), full((UL, U)),
            full((1, U)), full((1, U)), full((1, U)), full((1, U)),
            full((U, U)), full((1, U)), full((U, U)), full((1, U)),
            full((U, U)), full((1, U)), full((U, U)), full((1, U)),
        ],
        out_specs=[
            pl.BlockSpec((RTF, UL), lambda t: (t, 0)),
            full((2, UL)), full((BG, U)), full((BG, U)), full((BG, U)),
            full((BG, U)), full((BG, U)),
        ],
        out_shape=[
            jax.ShapeDtypeStruct((FR, UL), jnp.bfloat16),
            jax.ShapeDtypeStruct((2, UL), F32),
            jax.ShapeDtypeStruct((BG, U), F32),
            jax.ShapeDtypeStruct((BG, U), F32),
            jax.ShapeDtypeStruct((BG, U), F32),
            jax.ShapeDtypeStruct((BG, U), F32),
            jax.ShapeDtypeStruct((BG, U), F32),
        ],
        scratch_shapes=[
            pltpu.VMEM((BG, U), F32),
            pltpu.VMEM((1, UL), F32),
            pltpu.VMEM((1, UL), F32),
        ],
        compiler_params=pltpu.CompilerParams(
            dimension_semantics=("arbitrary",)),
    )

    apply_call = pl.pallas_call(
        _apply_body,
        grid=(T,),
        in_specs=[
            pl.BlockSpec((RTF, UL), lambda t: (t, 0)),
            pl.BlockSpec((RTF, UL), lambda t: (t, 0)),
            full((2, UL)),
        ],
        out_specs=pl.BlockSpec((RTF, UL), lambda t: (t, 0)),
        out_shape=jax.ShapeDtypeStruct((FR, UL), F32),
        compiler_params=pltpu.CompilerParams(
            dimension_semantics=("arbitrary",)),
    )

    x1, x3, t2, t4 = xproj_call(xc, v1_W[0], r1(v1_b[0]),
                                v2_W[0], r1(v2_b[0]),
                                v3_W[0], r1(v3_b[0]),
                                v4_W[0], r1(v4_b[0]))
    for i in range(DEPTH):
        g2, g4 = gather_call(t2, t4, flat_idx)
        g2 = g2.reshape(FR, UL)
        g4 = g4.reshape(FR, UL)
        E8 = jnp.kron(eye8, e0_W[i])
        j = (i + 1) % DEPTH
        u, scl, xn, x1n, x3n, t2n, t4n = stats_call(
            w, g2, g4, xc, x1, x3, E8, tile8(e0_b[i]), K, K2,
            r1(vbn_g[i]), r1(vbn_b[i]), r1(ebn_g[i]), r1(ebn_b[i]),
            v1_W[j], r1(v1_b[j]), v2_W[j], r1(v2_b[j]),
            v3_W[j], r1(v3_b[j]), v4_W[j], r1(v4_b[j]))
        w = apply_call(w, u, scl)
        xc, x1, x3, t2, t4 = xn, x1n, x3n, t2n, t4n

    return w.reshape(B, G, S, U)


# speedup vs baseline: 1.0438x; 1.0438x over previous
import functools

import jax
import jax.numpy as jnp
from jax import lax
from jax.experimental import pallas as pl
from jax.experimental.pallas import tpu as pltpu
import jax.experimental.pallas.tpu_sc as plsc

F32 = jnp.float32
FOLD = 8


def _init_body(xin, vW, vb, eW8, eb8, M8r, ea, x0_out, w0_out, *, RTF, U):
    t = pl.program_id(0)

    @pl.when(t == 0)
    def _():
        x0_out[...] = jax.nn.silu(
            jnp.dot(xin[...], vW[...], preferred_element_type=F32) + vb[...])

    eaf = jnp.dot(ea[...], M8r[...], preferred_element_type=F32)
    w0_out[...] = jax.nn.silu(eaf * eW8[...] + eb8[...])


def _xproj_body(x_ref, w1r, b1r, w2r, b2r, w3r, b3r, w4r, b4r,
                x1_out, x3_out, t2_out, t4_out):
    x = x_ref[...]
    x1_out[...] = jnp.dot(x, w1r[...], preferred_element_type=F32) + b1r[...]
    x3_out[...] = jnp.dot(x, w3r[...], preferred_element_type=F32) + b3r[...]
    t2_out[...] = jnp.dot(x, w2r[...], preferred_element_type=F32) + b2r[...]
    t4_out[...] = jnp.dot(x, w4r[...], preferred_element_type=F32) + b4r[...]


def _make_gather(BGS, U):
    NC, NS = 2, 16
    NW = NC * NS
    EPW = BGS // NW
    CH = 128
    NCH = EPW // CH
    NB = 4
    NGRP = NCH // NB
    SRPC = CH // FOLD
    UL = FOLD * U
    FR = BGS // FOLD
    mesh = plsc.VectorSubcoreMesh(core_axis_name="c", subcore_axis_name="s",
                                  num_cores=NC, num_subcores=NS)

    out_ty = jax.ShapeDtypeStruct((BGS, U), F32)
    scratch = [pltpu.VMEM((NCH, CH), jnp.int32)]
    scratch += [pltpu.VMEM((CH, U), F32) for _ in range(2 * NB)]
    scratch += [pltpu.SemaphoreType.DMA for _ in range(2 * NB)]

    @functools.partial(
        pl.kernel,
        out_type=[out_ty, out_ty],
        mesh=mesh,
        scratch_types=scratch,
        compiler_params=pltpu.CompilerParams(use_tc_tiling_on_sc=False),
    )
    def gather_k(t2_hbm, t4_hbm, idx_hbm, g2_hbm, g4_hbm, idx_v, *rest):
        r2 = rest[0:NB]
        r4 = rest[NB:2 * NB]
        gsem = rest[2 * NB:3 * NB]
        osem = rest[3 * NB:4 * NB]
        wid = lax.axis_index("s") * NC + lax.axis_index("c")
        obase = wid * EPW
        pltpu.sync_copy(idx_hbm.at[pl.ds(wid * NCH, NCH)], idx_v)

        def gath(c, b):
            pltpu.async_copy(t2_hbm.at[idx_v.at[c]], r2[b], gsem[b])
            pltpu.async_copy(t4_hbm.at[idx_v.at[c]], r4[b], gsem[b])

        def wait_g(b):
            pltpu.make_async_copy(t2_hbm.at[idx_v.at[0]], r2[b], gsem[b]).wait()
            pltpu.make_async_copy(t4_hbm.at[idx_v.at[0]], r4[b], gsem[b]).wait()

        g2f = g2_hbm
        g4f = g4_hbm

        def put(c, b):
            o = obase + c * CH
            pltpu.async_copy(r2[b], g2f.at[pl.ds(o, CH)], osem[b])
            pltpu.async_copy(r4[b], g4f.at[pl.ds(o, CH)], osem[b])

        def wait_o(b):
            pltpu.make_async_copy(r2[b], g2f.at[pl.ds(0, CH)],
                                  osem[b]).wait()
            pltpu.make_async_copy(r4[b], g4f.at[pl.ds(0, CH)],
                                  osem[b]).wait()

        for b in range(NB):
            gath(b, b)

        def grp_body(g, carry):
            for b in range(NB):
                c = g * NB + b
                wait_g(b)
                put(c, b)
                wait_o(b)
                gath(c + NB, b)
            return carry

        lax.fori_loop(0, NGRP - 1, grp_body, 0)
        for b in range(NB):
            wait_g(b)
            put((NGRP - 1) * NB + b, b)
        for b in range(NB):
            wait_o(b)

    return gather_k


def _stats_body(wp_ref, up_ref, sclp_ref, g2_ref, g4_ref,
                x_ref, x1_ref, x3_ref,
                E8r, b8r, Kr, K2r, Msegr, vgr, vbr, egr, ebr,
                w1n, b1n, w2n, b2n, w3n, b3n, w4n, b4n,
                w_out, u_out, scl_out, xn_out, x1n_out, x3n_out,
                t2n_out, t4n_out, agg, s1, s2, *, RTF, S, U, BGS, BG, T):
    t = pl.program_id(0)
    UL = FOLD * U
    NPT = RTF * FOLD // S
    SPF = S // FOLD

    @pl.when(t == 0)
    def _():
        s1[...] = jnp.zeros_like(s1)
        s2[...] = jnp.zeros_like(s2)

    up = up_ref[...].astype(F32)
    w0 = wp_ref[...] + jax.nn.silu(up * sclp_ref[0:1, :] + sclp_ref[1:2, :])
    w_out[...] = w0
    x3t = x3_ref[pl.ds(t * NPT, NPT), :]
    x3f8 = jnp.dot(x3t, Kr[...], preferred_element_type=F32)
    x3f = jnp.broadcast_to(x3f8[:, None, :],
                           (NPT, SPF, UL)).reshape(RTF, UL)
    w1 = jnp.dot(w0, E8r[...], preferred_element_type=F32) + b8r[...]
    u = w1 + x3f + g4_ref[...]
    ones_r = jnp.ones((1, RTF), F32)
    s1[...] += jnp.dot(ones_r, u, preferred_element_type=F32)
    s2[...] += jnp.dot(ones_r, u * u, preferred_element_type=F32)
    u_out[...] = u.astype(jnp.bfloat16)
    pr = jax.nn.sigmoid(w0) * g2_ref[...]
    pa = jnp.dot(Msegr[...], pr, preferred_element_type=F32)
    agg[pl.ds(t * NPT, NPT), :] = jnp.dot(
        pa, K2r[...], preferred_element_type=F32) / S

    @pl.when(t == T - 1)
    def _final():
        s1f = jnp.dot(s1[...], K2r[...], preferred_element_type=F32)
        s2f = jnp.dot(s2[...], K2r[...], preferred_element_type=F32)
        mean = s1f / BGS
        var = s2f / BGS - mean * mean
        inv = lax.rsqrt(var + 1e-5)
        sc48 = egr[...] * inv
        sh48 = ebr[...] - mean * sc48
        scl_out[...] = jnp.concatenate(
            [jnp.dot(sc48, Kr[...], preferred_element_type=F32),
             jnp.dot(sh48, Kr[...], preferred_element_type=F32)], axis=0)
        tv = x1_ref[...] + agg[...]
        mu = jnp.mean(tv, axis=0, keepdims=True)
        varv = jnp.mean(tv * tv, axis=0, keepdims=True) - mu * mu
        invv = lax.rsqrt(varv + 1e-5)
        xn = x_ref[...] + jax.nn.silu(
            (tv - mu) * invv * vgr[...] + vbr[...])
        xn_out[...] = xn
        x1n_out[...] = jnp.dot(xn, w1n[...], preferred_element_type=F32) + b1n[...]
        x3n_out[...] = jnp.dot(xn, w3n[...], preferred_element_type=F32) + b3n[...]
        t2n_out[...] = jnp.dot(xn, w2n[...], preferred_element_type=F32) + b2n[...]
        t4n_out[...] = jnp.dot(xn, w4n[...], preferred_element_type=F32) + b4n[...]


def _apply_body(w_ref, u_ref, scl_ref, wn_ref):
    u = u_ref[...].astype(F32)
    wn_ref[...] = w_ref[...] + jax.nn.silu(
        u * scl_ref[0:1, :] + scl_ref[1:2, :])


def kernel(x, edge_index, edge_attr, v_lin0_W, v_lin0_b, v1_W, v1_b,
           v2_W, v2_b, v3_W, v3_b, v4_W, v4_b, vbn_g, vbn_b,
           e_lin0_W, e_lin0_b, e0_W, e0_b, ebn_g, ebn_b):
    B, G, S = edge_index.shape
    U = v1_W.shape[-1]
    DEPTH = v1_W.shape[0]
    BG = B * G
    BGS = BG * S
    UL = FOLD * U
    FR = BGS // FOLD
    T = 16
    RTF = FR // T

    xin = x.reshape(BG, x.shape[-1])
    ea_f = edge_attr.reshape(FR, FOLD)
    flat_idx = (edge_index.astype(jnp.int32)
                + (jnp.arange(B, dtype=jnp.int32) * G)[:, None, None])
    flat_idx = flat_idx.reshape(BGS // 128, 128)

    r1 = lambda a: a.reshape(1, -1)
    tile8 = lambda a: jnp.tile(a.reshape(1, -1), (1, FOLD))
    eye8 = jnp.eye(FOLD, dtype=F32)

    init_call = pl.pallas_call(
        functools.partial(_init_body, RTF=RTF, U=U),
        grid=(T,),
        in_specs=[
            pl.BlockSpec((BG, xin.shape[1]), lambda t: (0, 0)),
            pl.BlockSpec(v_lin0_W.shape, lambda t: (0, 0)),
            pl.BlockSpec((1, U), lambda t: (0, 0)),
            pl.BlockSpec((1, UL), lambda t: (0, 0)),
            pl.BlockSpec((1, UL), lambda t: (0, 0)),
            pl.BlockSpec((FOLD, UL), lambda t: (0, 0)),
            pl.BlockSpec((RTF, FOLD), lambda t: (t, 0)),
        ],
        out_specs=[
            pl.BlockSpec((BG, U), lambda t: (0, 0)),
            pl.BlockSpec((RTF, UL), lambda t: (t, 0)),
        ],
        out_shape=[
            jax.ShapeDtypeStruct((BG, U), F32),
            jax.ShapeDtypeStruct((FR, UL), F32),
        ],
        compiler_params=pltpu.CompilerParams(
            dimension_semantics=("arbitrary",)),
    )
    M8 = jnp.kron(eye8, jnp.ones((1, U), dtype=F32))
    K = jnp.kron(jnp.ones((1, FOLD), dtype=F32), jnp.eye(U, dtype=F32))
    K2 = jnp.kron(jnp.ones((FOLD, 1), dtype=F32), jnp.eye(U, dtype=F32))
    xc, w = init_call(xin, v_lin0_W, r1(v_lin0_b),
                      tile8(e_lin0_W), tile8(e_lin0_b), M8, ea_f)

    xproj_call = pl.pallas_call(
        _xproj_body,
        out_shape=[jax.ShapeDtypeStruct((BG, U), F32) for _ in range(4)],
    )

    gather_call = _make_gather(BGS, U)

    full = lambda shp: pl.BlockSpec(shp, lambda t: (0, 0))
    stats_call = pl.pallas_call(
        functools.partial(_stats_body, RTF=RTF, S=S, U=U, BGS=BGS, BG=BG, T=T),
        grid=(T,),
        in_specs=[
            pl.BlockSpec((RTF, UL), lambda t: (t, 0)),
            pl.BlockSpec((RTF, UL), lambda t: (t, 0)),
            full((2, UL)),
            pl.BlockSpec((RTF, UL), lambda t: (t, 0)),
            pl.BlockSpec((RTF, UL), lambda t: (t, 0)),
            full((BG, U)), full((BG, U)), full((BG, U)),
            full((UL, UL)), full((1, UL)), full((U, UL)), full((UL, U)),
            full((BG // T, RTF)),
            full((1, U)), full((1, U)), full((1, U)), full((1, U)),
            full((U, U)), full((1, U)), full((U, U)), full((1, U)),
            full((U, U)), full((1, U)), full((U, U)), full((1, U)),
        ],
        out_specs=[
            pl.BlockSpec((RTF, UL), lambda t: (t, 0)),
            pl.BlockSpec((RTF, UL), lambda t: (t, 0)),
            full((2, UL)), full((BG, U)), full((BG, U)), full((BG, U)),
            full((BG, U)), full((BG, U)),
        ],
        out_shape=[
            jax.ShapeDtypeStruct((FR, UL), F32),
            jax.ShapeDtypeStruct((FR, UL), jnp.bfloat16),
            jax.ShapeDtypeStruct((2, UL), F32),
            jax.ShapeDtypeStruct((BG, U), F32),
            jax.ShapeDtypeStruct((BG, U), F32),
            jax.ShapeDtypeStruct((BG, U), F32),
            jax.ShapeDtypeStruct((BG, U), F32),
            jax.ShapeDtypeStruct((BG, U), F32),
        ],
        scratch_shapes=[
            pltpu.VMEM((BG, U), F32),
            pltpu.VMEM((1, UL), F32),
            pltpu.VMEM((1, UL), F32),
        ],
        compiler_params=pltpu.CompilerParams(
            dimension_semantics=("arbitrary",)),
    )

    apply_call = pl.pallas_call(
        _apply_body,
        grid=(T,),
        in_specs=[
            pl.BlockSpec((RTF, UL), lambda t: (t, 0)),
            pl.BlockSpec((RTF, UL), lambda t: (t, 0)),
            full((2, UL)),
        ],
        out_specs=pl.BlockSpec((RTF, UL), lambda t: (t, 0)),
        out_shape=jax.ShapeDtypeStruct((FR, UL), F32),
        compiler_params=pltpu.CompilerParams(
            dimension_semantics=("arbitrary",)),
    )

    x1, x3, t2, t4 = xproj_call(xc, v1_W[0], r1(v1_b[0]),
                                v2_W[0], r1(v2_b[0]),
                                v3_W[0], r1(v3_b[0]),
                                v4_W[0], r1(v4_b[0]))
    Mseg = jnp.kron(jnp.eye(BG // T, dtype=F32), jnp.ones((1, S // FOLD), F32))
    u = jnp.zeros((FR, UL), jnp.bfloat16)
    scl = jnp.zeros((2, UL), F32)
    for i in range(DEPTH):
        g2, g4 = gather_call(t2, t4, flat_idx)
        g2 = g2.reshape(FR, UL)
        g4 = g4.reshape(FR, UL)
        E8 = jnp.kron(eye8, e0_W[i])
        j = (i + 1) % DEPTH
        w, u, scl, xn, x1n, x3n, t2n, t4n = stats_call(
            w, u, scl, g2, g4, xc, x1, x3, E8, tile8(e0_b[i]), K, K2, Mseg,
            r1(vbn_g[i]), r1(vbn_b[i]), r1(ebn_g[i]), r1(ebn_b[i]),
            v1_W[j], r1(v1_b[j]), v2_W[j], r1(v2_b[j]),
            v3_W[j], r1(v3_b[j]), v4_W[j], r1(v4_b[j]))
        xc, x1, x3, t2, t4 = xn, x1n, x3n, t2n, t4n

    w = apply_call(w, u, scl)
    return w.reshape(B, G, S, U)
